# baseline (speedup 1.0000x reference)
import functools

import jax
import jax.numpy as jnp
from jax.experimental import pallas as pl
from jax.experimental.pallas import tpu as pltpu

_TEMP = 0.05
_BATCH = 1024
_FEAT = 128
_N = 100000
_TILE = 2048
_NBUF = 4
_NTILES = (_N + _TILE - 1) // _TILE
_LAST = _NTILES - 1
_REM = _N - _LAST * _TILE


def _copy(obuf, slot, out_hbm, col_start, width, sem):
    return pltpu.make_async_copy(
        obuf.at[slot, :, pl.ds(0, width)],
        out_hbm.at[:, pl.ds(col_start, width)],
        sem.at[slot],
    )


def _fused_kernel(x_ref, f_ref, tf_ref, dummy_ref, out_hbm, loss_ref,
                  obuf, last_buf, xn_ref, s_ref, sem, last_sem):
    j = pl.program_id(0)
    slot = jax.lax.rem(j, _NBUF)

    @pl.when(j == 0)
    def _init():
        x = x_ref[...]
        norm = jnp.sqrt(jnp.sum(x * x, axis=1, keepdims=True))
        xn_ref[...] = x / jnp.maximum(norm, 1e-12)
        s_ref[...] = jnp.zeros_like(s_ref)

    @pl.when(j >= _NBUF)
    def _reclaim():
        _copy(obuf, slot, out_hbm, (j - _NBUF) * _TILE, _TILE, sem).wait()

    xn = xn_ref[...]
    logits = jax.lax.dot_general(
        xn.astype(jnp.bfloat16), f_ref[...].astype(jnp.bfloat16),
        dimension_numbers=(((1,), (1,)), ((), ())),
        preferred_element_type=jnp.float32,
    ) * (1.0 / _TEMP)
    obuf[slot] = logits
    e = jnp.exp(logits)

    @pl.when(j < _LAST)
    def _stream():
        _copy(obuf, slot, out_hbm, j * _TILE, _TILE, sem).start()
        s_ref[...] += jnp.sum(e, axis=1, keepdims=True)

    @pl.when(j == _LAST)
    def _finish():
        last_buf[...] = logits[:, :_REM]
        last_copy = pltpu.make_async_copy(
            last_buf, out_hbm.at[:, pl.ds(_LAST * _TILE, _REM)], last_sem)
        last_copy.start()
        col = jax.lax.broadcasted_iota(jnp.int32, (_BATCH, _TILE), 1)
        e_last = jnp.where(col < _REM, e, 0.0)
        s = s_ref[...] + jnp.sum(e_last, axis=1, keepdims=True)
        tgt_logit = jnp.sum(xn * tf_ref[...], axis=1,
                            keepdims=True) * (1.0 / _TEMP)
        loss_ref[...] = jnp.mean(jnp.log(s) - tgt_logit).reshape(1, 1)
        for k in range(1, _NBUF):
            step = _LAST - k
            _copy(obuf, jax.lax.rem(jnp.int32(step), _NBUF), out_hbm,
                  step * _TILE, _TILE, sem).wait()
        last_copy.wait()


def kernel(inputs, targets, features):
    tgt_rows = jnp.take(features, targets.astype(jnp.int32), axis=0)
    dummy = jnp.zeros((_BATCH, _N), jnp.float32)

    outputs, loss = pl.pallas_call(
        _fused_kernel,
        grid=(_NTILES,),
        in_specs=[
            pl.BlockSpec((_BATCH, _FEAT), lambda j: (0, 0)),
            pl.BlockSpec((_TILE, _FEAT), lambda j: (j, 0)),
            pl.BlockSpec((_BATCH, _FEAT), lambda j: (0, 0)),
            pl.BlockSpec(memory_space=pl.ANY),
        ],
        input_output_aliases={3: 0},
        out_specs=[
            pl.BlockSpec(memory_space=pl.ANY),
            pl.BlockSpec((1, 1), lambda j: (0, 0)),
        ],
        out_shape=[
            jax.ShapeDtypeStruct((_BATCH, _N), jnp.float32),
            jax.ShapeDtypeStruct((1, 1), jnp.float32),
        ],
        scratch_shapes=[
            pltpu.VMEM((_NBUF, _BATCH, _TILE), jnp.float32),
            pltpu.VMEM((_BATCH, _REM), jnp.float32),
            pltpu.VMEM((_BATCH, _FEAT), jnp.float32),
            pltpu.VMEM((_BATCH, 1), jnp.float32),
            pltpu.SemaphoreType.DMA((_NBUF,)),
            pltpu.SemaphoreType.DMA,
        ],
    )(inputs, features, tgt_rows, dummy)

    return (loss.reshape(()), outputs)


# speedup vs baseline: 1.2332x; 1.2332x over previous
import functools

import jax
import jax.numpy as jnp
from jax.experimental import pallas as pl
from jax.experimental.pallas import tpu as pltpu

_TEMP = 0.05
_BATCH = 1024
_FEAT = 128
_N = 100000
_TILE = 2048
_NBUF = 4
_NTILES = (_N + _TILE - 1) // _TILE
_LAST = _NTILES - 1
_REM = _N - _LAST * _TILE


def _copy(obuf, slot, out_hbm, col_start, width, sem):
    return pltpu.make_async_copy(
        obuf.at[slot, :, pl.ds(0, width)],
        out_hbm.at[:, pl.ds(col_start, width)],
        sem.at[slot],
    )


def _fused_kernel(x_ref, f_ref, tf_ref, out_hbm, loss_ref,
                  obuf, last_buf, xn_ref, s_ref, sem, last_sem):
    j = pl.program_id(0)
    slot = jax.lax.rem(j, _NBUF)

    @pl.when(j == 0)
    def _init():
        x = x_ref[...]
        norm = jnp.sqrt(jnp.sum(x * x, axis=1, keepdims=True))
        xn_ref[...] = x / jnp.maximum(norm, 1e-12)
        s_ref[...] = jnp.zeros_like(s_ref)

    @pl.when(j >= _NBUF)
    def _reclaim():
        _copy(obuf, slot, out_hbm, (j - _NBUF) * _TILE, _TILE, sem).wait()

    xn = xn_ref[...]
    logits = jax.lax.dot_general(
        xn.astype(jnp.bfloat16), f_ref[...].astype(jnp.bfloat16),
        dimension_numbers=(((1,), (1,)), ((), ())),
        preferred_element_type=jnp.float32,
    ) * (1.0 / _TEMP)
    obuf[slot] = logits
    e = jnp.exp(logits)

    @pl.when(j < _LAST)
    def _stream():
        _copy(obuf, slot, out_hbm, j * _TILE, _TILE, sem).start()
        s_ref[...] += jnp.sum(e, axis=1, keepdims=True)

    @pl.when(j == _LAST)
    def _finish():
        last_buf[...] = logits[:, :_REM]
        last_copy = pltpu.make_async_copy(
            last_buf, out_hbm.at[:, pl.ds(_LAST * _TILE, _REM)], last_sem)
        last_copy.start()
        col = jax.lax.broadcasted_iota(jnp.int32, (_BATCH, _TILE), 1)
        e_last = jnp.where(col < _REM, e, 0.0)
        s = s_ref[...] + jnp.sum(e_last, axis=1, keepdims=True)
        tgt_logit = jnp.sum(xn * tf_ref[...], axis=1,
                            keepdims=True) * (1.0 / _TEMP)
        loss_ref[...] = jnp.mean(jnp.log(s) - tgt_logit).reshape(1, 1)
        for k in range(1, _NBUF):
            step = _LAST - k
            _copy(obuf, jax.lax.rem(jnp.int32(step), _NBUF), out_hbm,
                  step * _TILE, _TILE, sem).wait()
        last_copy.wait()


def kernel(inputs, targets, features):
    tgt_rows = jnp.take(features, targets.astype(jnp.int32), axis=0)

    outputs, loss = pl.pallas_call(
        _fused_kernel,
        grid=(_NTILES,),
        in_specs=[
            pl.BlockSpec((_BATCH, _FEAT), lambda j: (0, 0)),
            pl.BlockSpec((_TILE, _FEAT), lambda j: (j, 0)),
            pl.BlockSpec((_BATCH, _FEAT), lambda j: (0, 0)),
        ],
        out_specs=[
            pl.BlockSpec(memory_space=pl.ANY),
            pl.BlockSpec((1, 1), lambda j: (0, 0)),
        ],
        out_shape=[
            jax.ShapeDtypeStruct((_BATCH, _N), jnp.float32),
            jax.ShapeDtypeStruct((1, 1), jnp.float32),
        ],
        scratch_shapes=[
            pltpu.VMEM((_NBUF, _BATCH, _TILE), jnp.float32),
            pltpu.VMEM((_BATCH, _REM), jnp.float32),
            pltpu.VMEM((_BATCH, _FEAT), jnp.float32),
            pltpu.VMEM((_BATCH, 1), jnp.float32),
            pltpu.SemaphoreType.DMA((_NBUF,)),
            pltpu.SemaphoreType.DMA,
        ],
    )(inputs, features, tgt_rows)

    return (loss.reshape(()), outputs)


# speedup vs baseline: 1.2389x; 1.0046x over previous
import functools

import jax
import jax.numpy as jnp
from jax.experimental import pallas as pl
from jax.experimental.pallas import tpu as pltpu

_TEMP = 0.05
_BATCH = 1024
_FEAT = 128
_N = 100000
_TILE = 4096
_NTILES = (_N + _TILE - 1) // _TILE
_LAST = _NTILES - 1
_REM = _N - _LAST * _TILE


def _fused_kernel(x_ref, f_ref, tf_ref, out_ref, loss_ref, xn_ref, s_ref):
    j = pl.program_id(0)

    @pl.when(j == 0)
    def _init():
        x = x_ref[...]
        norm = jnp.sqrt(jnp.sum(x * x, axis=1, keepdims=True))
        xn_ref[...] = x / jnp.maximum(norm, 1e-12)
        s_ref[...] = jnp.zeros_like(s_ref)

    xn = xn_ref[...]
    logits = jax.lax.dot_general(
        xn.astype(jnp.bfloat16), f_ref[...].astype(jnp.bfloat16),
        dimension_numbers=(((1,), (1,)), ((), ())),
        preferred_element_type=jnp.float32,
    ) * (1.0 / _TEMP)
    out_ref[...] = logits
    e = jnp.exp(logits)

    @pl.when(j < _LAST)
    def _accum():
        s_ref[...] += jnp.sum(e, axis=1, keepdims=True)

    @pl.when(j == _LAST)
    def _finish():
        col = jax.lax.broadcasted_iota(jnp.int32, (_BATCH, _TILE), 1)
        e_last = jnp.where(col < _REM, e, 0.0)
        s = s_ref[...] + jnp.sum(e_last, axis=1, keepdims=True)
        tgt_logit = jnp.sum(xn * tf_ref[...], axis=1,
                            keepdims=True) * (1.0 / _TEMP)
        loss_ref[...] = jnp.mean(jnp.log(s) - tgt_logit).reshape(1, 1)


def kernel(inputs, targets, features):
    tgt_rows = jnp.take(features, targets.astype(jnp.int32), axis=0)

    outputs, loss = pl.pallas_call(
        _fused_kernel,
        grid=(_NTILES,),
        in_specs=[
            pl.BlockSpec((_BATCH, _FEAT), lambda j: (0, 0)),
            pl.BlockSpec((_TILE, _FEAT), lambda j: (j, 0)),
            pl.BlockSpec((_BATCH, _FEAT), lambda j: (0, 0)),
        ],
        out_specs=[
            pl.BlockSpec((_BATCH, _TILE), lambda j: (0, j)),
            pl.BlockSpec((1, 1), lambda j: (0, 0)),
        ],
        out_shape=[
            jax.ShapeDtypeStruct((_BATCH, _N), jnp.float32),
            jax.ShapeDtypeStruct((1, 1), jnp.float32),
        ],
        scratch_shapes=[
            pltpu.VMEM((_BATCH, _FEAT), jnp.float32),
            pltpu.VMEM((_BATCH, 1), jnp.float32),
        ],
    )(inputs, features, tgt_rows)

    return (loss.reshape(()), outputs)
